# baseline (speedup 1.0000x reference)
import functools

import jax
import jax.numpy as jnp
from jax import lax
from jax.experimental import pallas as pl
from jax.experimental.pallas import tpu as pltpu
from jax.experimental.pallas import tpu_sc as plsc

D = 1024
LANES = 16
DCHUNKS = D // LANES
TOK = 160
GCHUNK = 40
NROWS = 39
NC, NS = 2, 16
NW = NC * NS


def _segments():
    segs = [(0, 0, 24), (1, 24, 72)]
    for g in range(5):
        segs.append((2 + g, 96 + 3 * g, 3))
    for t in range(32):
        segs.append((7 + t, 111 + t, 1))
    return segs


def _chunk_pieces():
    bounds = [(0, 40), (40, 80), (80, 120), (120, 144)]
    chunks = []
    for lo, hi in bounds:
        pieces = []
        for row, s, c in _segments():
            a, b = max(s, lo), min(s + c, hi)
            if a < b:
                pieces.append((row, a - lo, b - a, a > s))
        chunks.append((lo, hi - lo, pieces))
    return chunks


_CHUNKS = _chunk_pieces()


def _sc_pool(tok, wgt, ftok, fwgt, table):
    bsz = tok.shape[0] // TOK
    per_w = bsz // NW
    mesh = plsc.VectorSubcoreMesh(core_axis_name="c", subcore_axis_name="s")

    @functools.partial(
        pl.kernel,
        mesh=mesh,
        out_type=(
            jax.ShapeDtypeStruct((bsz, NROWS, D), jnp.float32),
            jax.ShapeDtypeStruct((47, D), jnp.float32),
        ),
        scratch_types=[
            pltpu.VMEM((2, GCHUNK, D), jnp.float32),
            pltpu.VMEM((NROWS, D), jnp.float32),
            pltpu.VMEM((TOK,), jnp.int32),
            pltpu.VMEM((TOK,), jnp.float32),
            pltpu.VMEM((8,), jnp.int32),
            pltpu.VMEM((16,), jnp.float32),
            pltpu.VMEM((D,), jnp.float32),
            pltpu.SemaphoreType.DMA,
            pltpu.SemaphoreType.DMA,
        ],
    )
    def pool(tok_hbm, w_hbm, ftok_hbm, fw_hbm, table_hbm, vals_hbm, fname_hbm,
             stage2_v, out_v, idx_v, wv_v, fidx_v, fw_v, frow_v, sem,
             osem):
        wid = lax.axis_index("s") * NC + lax.axis_index("c")

        def do_elem(i, carry):
            b = wid * per_w + i
            pltpu.sync_copy(tok_hbm.at[pl.ds(b * TOK, TOK)], idx_v)
            pltpu.sync_copy(w_hbm.at[pl.ds(b * TOK, TOK)], wv_v)
            lo0, cnt0, _ = _CHUNKS[0]
            handles = {0: pltpu.async_copy(
                table_hbm.at[idx_v.at[pl.ds(lo0, cnt0)]],
                stage2_v.at[0].at[pl.ds(0, cnt0)], sem)}
            for ci, (lo, cnt, pieces) in enumerate(_CHUNKS):
                handles[ci].wait()
                if ci + 1 < len(_CHUNKS):
                    nlo, ncnt, _ = _CHUNKS[ci + 1]
                    handles[ci + 1] = pltpu.async_copy(
                        table_hbm.at[idx_v.at[pl.ds(nlo, ncnt)]],
                        stage2_v.at[(ci + 1) % 2].at[pl.ds(0, ncnt)], sem)
                stage_v = stage2_v.at[ci % 2]
                wvec = {base: wv_v[pl.ds(base, LANES)]
                        for base in range((lo // LANES) * LANES,
                                          lo + cnt, LANES)}
                wsc = {}
                for (row, ls, n, accum) in pieces:
                    for j in range(n):
                        t = lo + ls + j
                        wsc[t] = wvec[(t // LANES) * LANES][t % LANES]

                def chunk_body(c, carry2, lo=lo, pieces=pieces, wsc=wsc,
                               stage_v=stage_v):
                    for half in range(2):
                        sl = pl.ds((2 * c + half) * LANES, LANES)
                        for (row, ls, n, accum) in pieces:
                            acc = stage_v[ls, sl] * wsc[lo + ls]
                            for j in range(1, n):
                                acc = acc + (stage_v[ls + j, sl]
                                             * wsc[lo + ls + j])
                            if accum:
                                out_v[row, sl] = out_v[row, sl] + acc
                            else:
                                out_v[row, sl] = acc
                    return carry2

                lax.fori_loop(0, DCHUNKS // 2, chunk_body, 0)
            pltpu.async_copy(out_v, vals_hbm.at[b], osem).wait()
            return carry

        lax.fori_loop(0, per_w, do_elem, 0)

        for off in (0, 32):
            row = wid + off

            @pl.when(row < 47)
            def _():
                pltpu.sync_copy(ftok_hbm.at[pl.ds(row * 8, 8)], fidx_v)
                pltpu.sync_copy(fw_hbm.at[pl.ds(row * 8, 8)],
                                fw_v.at[pl.ds(0, 8)])
                fstage = stage2_v.at[0]
                pltpu.async_copy(
                    table_hbm.at[fidx_v], fstage.at[pl.ds(0, 8)], sem).wait()
                fvec = fw_v[...]
                fsc = [fvec[j] for j in range(6)]

                def fbody(c, carry2):
                    sl = pl.ds(c * LANES, LANES)
                    acc = fstage[0, sl] * fsc[0]
                    for j in range(1, 6):
                        acc = acc + fstage[j, sl] * fsc[j]
                    frow_v[sl] = acc
                    return carry2

                lax.fori_loop(0, DCHUNKS, fbody, 0)
                pltpu.sync_copy(frow_v, fname_hbm.at[row])

    return pool(tok, wgt, ftok, fwgt, table)


def _names_fc(field_name, w_top, b_fc):
    def body(f_ref, w_ref, b_ref, o_ref):
        bf = jnp.bfloat16
        o_ref[...] = jnp.dot(
            f_ref[...].astype(bf), w_ref[...].astype(bf),
            preferred_element_type=jnp.float32) + b_ref[...][None]

    return pl.pallas_call(
        body,
        out_shape=jax.ShapeDtypeStruct((47, D), jnp.float32),
    )(field_name, w_top, b_fc)


def _fc(vals, rating, hours, names_fc, w_rating, w_hours, w_bot, w_lin):
    bsz = vals.shape[0]
    bt = 32
    bf = jnp.bfloat16

    def body(v_ref, r_ref, h_ref, nf_ref, wr_ref, wh_ref, wb_ref, wl_ref,
             o_ref):
        v = v_ref[...]
        re = jnp.dot(r_ref[...].astype(bf), wr_ref[...].astype(bf),
                     preferred_element_type=jnp.float32)
        he = jnp.dot(h_ref[...].astype(bf).reshape(bt * 7, 4),
                     wh_ref[...].astype(bf),
                     preferred_element_type=jnp.float32)
        x = jnp.concatenate(
            [v, re[:, None, :], he.reshape(bt, 7, D)], axis=1)
        fc = jnp.dot(x.astype(bf).reshape(bt * 47, D), wb_ref[...].astype(bf),
                     preferred_element_type=jnp.float32)
        fc = fc.reshape(bt, 47, D) + nf_ref[...][None]
        fc = jnp.maximum(fc, 0.0)
        out = jnp.dot(fc.astype(bf).reshape(bt * 47, D),
                      wl_ref[...].astype(bf),
                      preferred_element_type=jnp.float32)
        o_ref[...] = out.reshape(bt, 47, D)

    return pl.pallas_call(
        body,
        grid=(bsz // bt,),
        in_specs=[
            pl.BlockSpec((bt, NROWS, D), lambda i: (i, 0, 0)),
            pl.BlockSpec((bt, 4), lambda i: (i, 0)),
            pl.BlockSpec((bt, 7, 4), lambda i: (i, 0, 0)),
            pl.BlockSpec((47, D), lambda i: (0, 0)),
            pl.BlockSpec((4, D), lambda i: (0, 0)),
            pl.BlockSpec((4, D), lambda i: (0, 0)),
            pl.BlockSpec((D, D), lambda i: (0, 0)),
            pl.BlockSpec((D, D), lambda i: (0, 0)),
        ],
        out_specs=pl.BlockSpec((bt, 47, D), lambda i: (i, 0, 0)),
        out_shape=jax.ShapeDtypeStruct((bsz, 47, D), jnp.float32),
    )(vals, rating, hours, names_fc, w_rating, w_hours, w_bot, w_lin)


def kernel(field, name, category, str_categorical, str_boolean, rating, hours,
           emb_table, W_rating, W_hours, W_fc, b_fc, W_lin):
    bsz = name.shape[0]
    f32 = jnp.float32

    nm = (name != 1).astype(f32)
    cm = (category != 1).astype(f32)
    gmask = cm.max(axis=-1)
    denom = gmask.sum(axis=-1, keepdims=True) + 1e-6
    wcat = cm * (gmask / denom)[..., None]
    scm = (str_categorical != 1).astype(f32)
    sbm = (str_boolean[..., 0] != 1).astype(f32)

    tok = jnp.concatenate(
        [name, category.reshape(bsz, 72), str_categorical.reshape(bsz, 15),
         str_boolean[..., 0]], axis=1)
    tok = jnp.pad(tok, ((0, 0), (0, TOK - 143))).astype(jnp.int32)
    wgt = jnp.concatenate(
        [nm, wcat.reshape(bsz, 72), scm.reshape(bsz, 15), sbm], axis=1)
    wgt = jnp.pad(wgt, ((0, 0), (0, TOK - 143)))

    ftok = jnp.pad(field, ((0, 0), (0, 2))).astype(jnp.int32)
    fwgt = jnp.pad((field != 1).astype(f32), ((0, 0), (0, 2)))

    vals, field_name = _sc_pool(tok.reshape(-1), wgt.reshape(-1),
                                ftok.reshape(-1), fwgt.reshape(-1), emb_table)
    names_fc = _names_fc(field_name, W_fc[:D], b_fc)
    out = _fc(vals, rating, hours, names_fc, W_rating, W_hours, W_fc[D:],
              W_lin)

    name_mask = jnp.ones((bsz, 1), dtype=bool)
    category_mask = category[:, :1, 0] != 1
    str_categorical_mask = str_categorical[:, :, 0] != 1
    str_boolean_mask = str_boolean[:, :, 0] != 1
    rating_mask = jnp.ones((bsz, 1), dtype=bool)
    hours_mask = hours.sum(axis=-1) != 0.0
    all_masks = jnp.concatenate(
        [name_mask, category_mask, str_categorical_mask, str_boolean_mask,
         rating_mask, hours_mask], axis=1)
    return out, all_masks


# speedup vs baseline: 1.3960x; 1.3960x over previous
import functools

import jax
import jax.numpy as jnp
from jax import lax
from jax.experimental import pallas as pl
from jax.experimental.pallas import tpu as pltpu
from jax.experimental.pallas import tpu_sc as plsc

D = 1024
LANES = 16
DCHUNKS = D // LANES
TOK = 160
GCHUNK = 40
NROWS = 39
NC, NS = 2, 16
NW = NC * NS


def _segments():
    segs = [(0, 0, 24), (1, 24, 72)]
    for g in range(5):
        segs.append((2 + g, 96 + 3 * g, 3))
    for t in range(32):
        segs.append((7 + t, 111 + t, 1))
    return segs


def _chunk_pieces():
    bounds = [(0, 40), (40, 80), (80, 120), (120, 144)]
    chunks = []
    for lo, hi in bounds:
        pieces = []
        for row, s, c in _segments():
            a, b = max(s, lo), min(s + c, hi)
            if a < b:
                pieces.append((row, a - lo, b - a, a > s))
        chunks.append((lo, hi - lo, pieces))
    return chunks


_CHUNKS = _chunk_pieces()


def _sc_pool(tok, wgt, ftok, fwgt, table):
    bsz = tok.shape[0] // TOK
    per_w = bsz // NW
    mesh = plsc.VectorSubcoreMesh(core_axis_name="c", subcore_axis_name="s")

    @functools.partial(
        pl.kernel,
        mesh=mesh,
        out_type=(
            jax.ShapeDtypeStruct((bsz, NROWS, D), jnp.float32),
            jax.ShapeDtypeStruct((47, D), jnp.float32),
        ),
        scratch_types=[
            pltpu.VMEM((2, GCHUNK, D), jnp.float32),
            pltpu.VMEM((NROWS, D), jnp.float32),
            pltpu.VMEM((TOK,), jnp.int32),
            pltpu.VMEM((TOK,), jnp.float32),
            pltpu.VMEM((8,), jnp.int32),
            pltpu.VMEM((16,), jnp.float32),
            pltpu.VMEM((D,), jnp.float32),
            pltpu.SemaphoreType.DMA,
            pltpu.SemaphoreType.DMA,
        ],
    )
    def pool(tok_hbm, w_hbm, ftok_hbm, fw_hbm, table_hbm, vals_hbm, fname_hbm,
             stage2_v, out_v, idx_v, wv_v, fidx_v, fw_v, frow_v, sem,
             osem):
        wid = lax.axis_index("s") * NC + lax.axis_index("c")

        def do_elem(i, carry):
            b = wid * per_w + i
            pltpu.sync_copy(tok_hbm.at[pl.ds(b * TOK, TOK)], idx_v)
            pltpu.sync_copy(w_hbm.at[pl.ds(b * TOK, TOK)], wv_v)
            lo0, cnt0, _ = _CHUNKS[0]
            handles = {0: pltpu.async_copy(
                table_hbm.at[idx_v.at[pl.ds(lo0, cnt0)]],
                stage2_v.at[0].at[pl.ds(0, cnt0)], sem)}
            for ci, (lo, cnt, pieces) in enumerate(_CHUNKS):
                handles[ci].wait()
                if ci + 1 < len(_CHUNKS):
                    nlo, ncnt, _ = _CHUNKS[ci + 1]
                    handles[ci + 1] = pltpu.async_copy(
                        table_hbm.at[idx_v.at[pl.ds(nlo, ncnt)]],
                        stage2_v.at[(ci + 1) % 2].at[pl.ds(0, ncnt)], sem)
                stage_v = stage2_v.at[ci % 2]
                wvec = {base: wv_v[pl.ds(base, LANES)]
                        for base in range((lo // LANES) * LANES,
                                          lo + cnt, LANES)}
                wsc = {}
                for (row, ls, n, accum) in pieces:
                    for j in range(n):
                        t = lo + ls + j
                        wsc[t] = wvec[(t // LANES) * LANES][t % LANES]

                def chunk_body(c, carry2, lo=lo, pieces=pieces, wsc=wsc,
                               stage_v=stage_v):
                    sl = pl.ds(c * LANES, LANES)
                    for (row, ls, n, accum) in pieces:
                        acc = stage_v[ls, sl] * wsc[lo + ls]
                        for j in range(1, n):
                            acc = acc + stage_v[ls + j, sl] * wsc[lo + ls + j]
                        if accum:
                            out_v[row, sl] = out_v[row, sl] + acc
                        else:
                            out_v[row, sl] = acc
                    return carry2

                lax.fori_loop(0, DCHUNKS, chunk_body, 0)
            pltpu.async_copy(out_v, vals_hbm.at[b], osem).wait()
            return carry

        lax.fori_loop(0, per_w, do_elem, 0)

        for off in (0, 32):
            row = wid + off

            @pl.when(row < 47)
            def _():
                pltpu.sync_copy(ftok_hbm.at[pl.ds(row * 8, 8)], fidx_v)
                pltpu.sync_copy(fw_hbm.at[pl.ds(row * 8, 8)],
                                fw_v.at[pl.ds(0, 8)])
                fstage = stage2_v.at[0]
                pltpu.async_copy(
                    table_hbm.at[fidx_v], fstage.at[pl.ds(0, 8)], sem).wait()
                fvec = fw_v[...]
                fsc = [fvec[j] for j in range(6)]

                def fbody(c, carry2):
                    sl = pl.ds(c * LANES, LANES)
                    acc = fstage[0, sl] * fsc[0]
                    for j in range(1, 6):
                        acc = acc + fstage[j, sl] * fsc[j]
                    frow_v[sl] = acc
                    return carry2

                lax.fori_loop(0, DCHUNKS, fbody, 0)
                pltpu.sync_copy(frow_v, fname_hbm.at[row])

    return pool(tok, wgt, ftok, fwgt, table)


def _names_fc(field_name, w_top, b_fc):
    def body(f_ref, w_ref, b_ref, o_ref):
        bf = jnp.bfloat16
        o_ref[...] = jnp.dot(
            f_ref[...].astype(bf), w_ref[...].astype(bf),
            preferred_element_type=jnp.float32) + b_ref[...][None]

    return pl.pallas_call(
        body,
        out_shape=jax.ShapeDtypeStruct((47, D), jnp.float32),
    )(field_name, w_top, b_fc)


def _fc(vals, rating, hours, names_fc, w_rating, w_hours, w_bot, w_lin):
    bsz = vals.shape[0]
    bt = 32
    bf = jnp.bfloat16

    def body(v_ref, r_ref, h_ref, nf_ref, wr_ref, wh_ref, wb_ref, wl_ref,
             o_ref):
        v = v_ref[...]
        re = jnp.dot(r_ref[...].astype(bf), wr_ref[...].astype(bf),
                     preferred_element_type=jnp.float32)
        he = jnp.dot(h_ref[...].astype(bf).reshape(bt * 7, 4),
                     wh_ref[...].astype(bf),
                     preferred_element_type=jnp.float32)
        x = jnp.concatenate(
            [v, re[:, None, :], he.reshape(bt, 7, D)], axis=1)
        fc = jnp.dot(x.astype(bf).reshape(bt * 47, D), wb_ref[...].astype(bf),
                     preferred_element_type=jnp.float32)
        fc = fc.reshape(bt, 47, D) + nf_ref[...][None]
        fc = jnp.maximum(fc, 0.0)
        out = jnp.dot(fc.astype(bf).reshape(bt * 47, D),
                      wl_ref[...].astype(bf),
                      preferred_element_type=jnp.float32)
        o_ref[...] = out.reshape(bt, 47, D)

    return pl.pallas_call(
        body,
        grid=(bsz // bt,),
        in_specs=[
            pl.BlockSpec((bt, NROWS, D), lambda i: (i, 0, 0)),
            pl.BlockSpec((bt, 4), lambda i: (i, 0)),
            pl.BlockSpec((bt, 7, 4), lambda i: (i, 0, 0)),
            pl.BlockSpec((47, D), lambda i: (0, 0)),
            pl.BlockSpec((4, D), lambda i: (0, 0)),
            pl.BlockSpec((4, D), lambda i: (0, 0)),
            pl.BlockSpec((D, D), lambda i: (0, 0)),
            pl.BlockSpec((D, D), lambda i: (0, 0)),
        ],
        out_specs=pl.BlockSpec((bt, 47, D), lambda i: (i, 0, 0)),
        out_shape=jax.ShapeDtypeStruct((bsz, 47, D), jnp.float32),
    )(vals, rating, hours, names_fc, w_rating, w_hours, w_bot, w_lin)


def kernel(field, name, category, str_categorical, str_boolean, rating, hours,
           emb_table, W_rating, W_hours, W_fc, b_fc, W_lin):
    bsz = name.shape[0]
    f32 = jnp.float32

    nm = (name != 1).astype(f32)
    cm = (category != 1).astype(f32)
    gmask = cm.max(axis=-1)
    denom = gmask.sum(axis=-1, keepdims=True) + 1e-6
    wcat = cm * (gmask / denom)[..., None]
    scm = (str_categorical != 1).astype(f32)
    sbm = (str_boolean[..., 0] != 1).astype(f32)

    tok = jnp.concatenate(
        [name, category.reshape(bsz, 72), str_categorical.reshape(bsz, 15),
         str_boolean[..., 0]], axis=1)
    tok = jnp.pad(tok, ((0, 0), (0, TOK - 143))).astype(jnp.int32)
    wgt = jnp.concatenate(
        [nm, wcat.reshape(bsz, 72), scm.reshape(bsz, 15), sbm], axis=1)
    wgt = jnp.pad(wgt, ((0, 0), (0, TOK - 143)))

    ftok = jnp.pad(field, ((0, 0), (0, 2))).astype(jnp.int32)
    fwgt = jnp.pad((field != 1).astype(f32), ((0, 0), (0, 2)))

    vals, field_name = _sc_pool(tok.reshape(-1), wgt.reshape(-1),
                                ftok.reshape(-1), fwgt.reshape(-1), emb_table)
    names_fc = _names_fc(field_name, W_fc[:D], b_fc)
    out = _fc(vals, rating, hours, names_fc, W_rating, W_hours, W_fc[D:],
              W_lin)

    name_mask = jnp.ones((bsz, 1), dtype=bool)
    category_mask = category[:, :1, 0] != 1
    str_categorical_mask = str_categorical[:, :, 0] != 1
    str_boolean_mask = str_boolean[:, :, 0] != 1
    rating_mask = jnp.ones((bsz, 1), dtype=bool)
    hours_mask = hours.sum(axis=-1) != 0.0
    all_masks = jnp.concatenate(
        [name_mask, category_mask, str_categorical_mask, str_boolean_mask,
         rating_mask, hours_mask], axis=1)
    return out, all_masks


# speedup vs baseline: 1.4572x; 1.0438x over previous
import functools

import jax
import jax.numpy as jnp
from jax import lax
from jax.experimental import pallas as pl
from jax.experimental.pallas import tpu as pltpu
from jax.experimental.pallas import tpu_sc as plsc

D = 1024
LANES = 16
DCHUNKS = D // LANES
TOK = 160
GCHUNK = 40
NROWS = 39
NC, NS = 2, 16
NW = NC * NS


def _segments():
    segs = [(0, 0, 24), (1, 24, 72)]
    for g in range(5):
        segs.append((2 + g, 96 + 3 * g, 3))
    for t in range(32):
        segs.append((7 + t, 111 + t, 1))
    return segs


def _chunk_pieces():
    bounds = [(0, 40), (40, 80), (80, 120), (120, 144)]
    chunks = []
    for lo, hi in bounds:
        pieces = []
        for row, s, c in _segments():
            a, b = max(s, lo), min(s + c, hi)
            if a < b:
                pieces.append((row, a - lo, b - a, a > s))
        chunks.append((lo, hi - lo, pieces))
    return chunks


_CHUNKS = _chunk_pieces()


def _sc_pool(tok, wgt, ftok, fwgt, table):
    bsz = tok.shape[0] // TOK
    per_w = bsz // NW
    mesh = plsc.VectorSubcoreMesh(core_axis_name="c", subcore_axis_name="s")

    @functools.partial(
        pl.kernel,
        mesh=mesh,
        out_type=(
            jax.ShapeDtypeStruct((bsz, NROWS, D), jnp.float32),
            jax.ShapeDtypeStruct((47, D), jnp.float32),
        ),
        scratch_types=[
            pltpu.VMEM((2, GCHUNK, D), jnp.float32),
            pltpu.VMEM((NROWS, D), jnp.float32),
            pltpu.VMEM((TOK,), jnp.int32),
            pltpu.VMEM((TOK,), jnp.float32),
            pltpu.VMEM((8,), jnp.int32),
            pltpu.VMEM((16,), jnp.float32),
            pltpu.VMEM((D,), jnp.float32),
            pltpu.SemaphoreType.DMA,
            pltpu.SemaphoreType.DMA,
        ],
    )
    def pool(tok_hbm, w_hbm, ftok_hbm, fw_hbm, table_hbm, vals_hbm, fname_hbm,
             stage2_v, out_v, idx_v, wv_v, fidx_v, fw_v, frow_v, sem,
             osem):
        wid = lax.axis_index("s") * NC + lax.axis_index("c")

        def do_elem(i, carry):
            b = wid * per_w + i
            pltpu.sync_copy(tok_hbm.at[pl.ds(b * TOK, TOK)], idx_v)
            pltpu.sync_copy(w_hbm.at[pl.ds(b * TOK, TOK)], wv_v)
            lo0, cnt0, _ = _CHUNKS[0]
            handles = {0: pltpu.async_copy(
                table_hbm.at[idx_v.at[pl.ds(lo0, cnt0)]],
                stage2_v.at[0].at[pl.ds(0, cnt0)], sem)}
            for ci, (lo, cnt, pieces) in enumerate(_CHUNKS):
                handles[ci].wait()
                if ci + 1 < len(_CHUNKS):
                    nlo, ncnt, _ = _CHUNKS[ci + 1]
                    handles[ci + 1] = pltpu.async_copy(
                        table_hbm.at[idx_v.at[pl.ds(nlo, ncnt)]],
                        stage2_v.at[(ci + 1) % 2].at[pl.ds(0, ncnt)], sem)
                stage_v = stage2_v.at[ci % 2]
                if ci == 0:
                    @pl.when(i > 0)
                    def _():
                        pltpu.make_async_copy(
                            vals_hbm.at[b], out_v, osem).wait()
                wvec = {base: wv_v[pl.ds(base, LANES)]
                        for base in range((lo // LANES) * LANES,
                                          lo + cnt, LANES)}
                wsc = {}
                for (row, ls, n, accum) in pieces:
                    for j in range(n):
                        t = lo + ls + j
                        wsc[t] = wvec[(t // LANES) * LANES][t % LANES]

                def chunk_body(c, carry2, lo=lo, pieces=pieces, wsc=wsc,
                               stage_v=stage_v):
                    sl = pl.ds(c * LANES, LANES)
                    for (row, ls, n, accum) in pieces:
                        acc = stage_v[ls, sl] * wsc[lo + ls]
                        for j in range(1, n):
                            acc = acc + stage_v[ls + j, sl] * wsc[lo + ls + j]
                        if accum:
                            out_v[row, sl] = out_v[row, sl] + acc
                        else:
                            out_v[row, sl] = acc
                    return carry2

                lax.fori_loop(0, DCHUNKS, chunk_body, 0)
            pltpu.async_copy(out_v, vals_hbm.at[b], osem)
            return carry

        lax.fori_loop(0, per_w, do_elem, 0)
        pltpu.make_async_copy(vals_hbm.at[0], out_v, osem).wait()

        for off in (0, 32):
            row = wid + off

            @pl.when(row < 47)
            def _():
                pltpu.sync_copy(ftok_hbm.at[pl.ds(row * 8, 8)], fidx_v)
                pltpu.sync_copy(fw_hbm.at[pl.ds(row * 8, 8)],
                                fw_v.at[pl.ds(0, 8)])
                fstage = stage2_v.at[0]
                pltpu.async_copy(
                    table_hbm.at[fidx_v], fstage.at[pl.ds(0, 8)], sem).wait()
                fvec = fw_v[...]
                fsc = [fvec[j] for j in range(6)]

                def fbody(c, carry2):
                    sl = pl.ds(c * LANES, LANES)
                    acc = fstage[0, sl] * fsc[0]
                    for j in range(1, 6):
                        acc = acc + fstage[j, sl] * fsc[j]
                    frow_v[sl] = acc
                    return carry2

                lax.fori_loop(0, DCHUNKS, fbody, 0)
                pltpu.sync_copy(frow_v, fname_hbm.at[row])

    return pool(tok, wgt, ftok, fwgt, table)


def _names_fc(field_name, w_top, b_fc):
    def body(f_ref, w_ref, b_ref, o_ref):
        bf = jnp.bfloat16
        o_ref[...] = jnp.dot(
            f_ref[...].astype(bf), w_ref[...].astype(bf),
            preferred_element_type=jnp.float32) + b_ref[...][None]

    return pl.pallas_call(
        body,
        out_shape=jax.ShapeDtypeStruct((47, D), jnp.float32),
    )(field_name, w_top, b_fc)


def _fc(vals, rating, hours, names_fc, w_rating, w_hours, w_bot, w_lin):
    bsz = vals.shape[0]
    bt = 32
    bf = jnp.bfloat16

    def body(v_ref, r_ref, h_ref, nf_ref, wr_ref, wh_ref, wb_ref, wl_ref,
             o_ref):
        v = v_ref[...]
        re = jnp.dot(r_ref[...].astype(bf), wr_ref[...].astype(bf),
                     preferred_element_type=jnp.float32)
        he = jnp.dot(h_ref[...].astype(bf).reshape(bt * 7, 4),
                     wh_ref[...].astype(bf),
                     preferred_element_type=jnp.float32)
        x = jnp.concatenate(
            [v, re[:, None, :], he.reshape(bt, 7, D)], axis=1)
        fc = jnp.dot(x.astype(bf).reshape(bt * 47, D), wb_ref[...].astype(bf),
                     preferred_element_type=jnp.float32)
        fc = fc.reshape(bt, 47, D) + nf_ref[...][None]
        fc = jnp.maximum(fc, 0.0)
        out = jnp.dot(fc.astype(bf).reshape(bt * 47, D),
                      wl_ref[...].astype(bf),
                      preferred_element_type=jnp.float32)
        o_ref[...] = out.reshape(bt, 47, D)

    return pl.pallas_call(
        body,
        grid=(bsz // bt,),
        in_specs=[
            pl.BlockSpec((bt, NROWS, D), lambda i: (i, 0, 0)),
            pl.BlockSpec((bt, 4), lambda i: (i, 0)),
            pl.BlockSpec((bt, 7, 4), lambda i: (i, 0, 0)),
            pl.BlockSpec((47, D), lambda i: (0, 0)),
            pl.BlockSpec((4, D), lambda i: (0, 0)),
            pl.BlockSpec((4, D), lambda i: (0, 0)),
            pl.BlockSpec((D, D), lambda i: (0, 0)),
            pl.BlockSpec((D, D), lambda i: (0, 0)),
        ],
        out_specs=pl.BlockSpec((bt, 47, D), lambda i: (i, 0, 0)),
        out_shape=jax.ShapeDtypeStruct((bsz, 47, D), jnp.float32),
    )(vals, rating, hours, names_fc, w_rating, w_hours, w_bot, w_lin)


def kernel(field, name, category, str_categorical, str_boolean, rating, hours,
           emb_table, W_rating, W_hours, W_fc, b_fc, W_lin):
    bsz = name.shape[0]
    f32 = jnp.float32

    nm = (name != 1).astype(f32)
    cm = (category != 1).astype(f32)
    gmask = cm.max(axis=-1)
    denom = gmask.sum(axis=-1, keepdims=True) + 1e-6
    wcat = cm * (gmask / denom)[..., None]
    scm = (str_categorical != 1).astype(f32)
    sbm = (str_boolean[..., 0] != 1).astype(f32)

    tok = jnp.concatenate(
        [name, category.reshape(bsz, 72), str_categorical.reshape(bsz, 15),
         str_boolean[..., 0]], axis=1)
    tok = jnp.pad(tok, ((0, 0), (0, TOK - 143))).astype(jnp.int32)
    wgt = jnp.concatenate(
        [nm, wcat.reshape(bsz, 72), scm.reshape(bsz, 15), sbm], axis=1)
    wgt = jnp.pad(wgt, ((0, 0), (0, TOK - 143)))

    ftok = jnp.pad(field, ((0, 0), (0, 2))).astype(jnp.int32)
    fwgt = jnp.pad((field != 1).astype(f32), ((0, 0), (0, 2)))

    vals, field_name = _sc_pool(tok.reshape(-1), wgt.reshape(-1),
                                ftok.reshape(-1), fwgt.reshape(-1), emb_table)
    names_fc = _names_fc(field_name, W_fc[:D], b_fc)
    out = _fc(vals, rating, hours, names_fc, W_rating, W_hours, W_fc[D:],
              W_lin)

    name_mask = jnp.ones((bsz, 1), dtype=bool)
    category_mask = category[:, :1, 0] != 1
    str_categorical_mask = str_categorical[:, :, 0] != 1
    str_boolean_mask = str_boolean[:, :, 0] != 1
    rating_mask = jnp.ones((bsz, 1), dtype=bool)
    hours_mask = hours.sum(axis=-1) != 0.0
    all_masks = jnp.concatenate(
        [name_mask, category_mask, str_categorical_mask, str_boolean_mask,
         rating_mask, hours_mask], axis=1)
    return out, all_masks


# speedup vs baseline: 1.5090x; 1.0356x over previous
import functools

import jax
import jax.numpy as jnp
from jax import lax
from jax.experimental import pallas as pl
from jax.experimental.pallas import tpu as pltpu
from jax.experimental.pallas import tpu_sc as plsc

D = 1024
LANES = 16
DCHUNKS = D // LANES
TOK = 160
GCHUNK = 40
NROWS = 39
NC, NS = 2, 16
NW = NC * NS


def _segments():
    segs = [(0, 0, 24), (1, 24, 72)]
    for g in range(5):
        segs.append((2 + g, 96 + 3 * g, 3))
    for t in range(32):
        segs.append((7 + t, 111 + t, 1))
    return segs


def _chunk_pieces():
    bounds = [(0, 40), (40, 80), (80, 120), (120, 144)]
    chunks = []
    for lo, hi in bounds:
        pieces = []
        for row, s, c in _segments():
            a, b = max(s, lo), min(s + c, hi)
            if a < b:
                pieces.append((row, a - lo, b - a, a > s))
        chunks.append((lo, hi - lo, pieces))
    return chunks


_CHUNKS = _chunk_pieces()


def _sc_pool(tok, wgt, ftok, fwgt, table):
    bsz = tok.shape[0] // TOK
    per_w = bsz // NW
    mesh = plsc.VectorSubcoreMesh(core_axis_name="c", subcore_axis_name="s")

    @functools.partial(
        pl.kernel,
        mesh=mesh,
        out_type=(
            jax.ShapeDtypeStruct((bsz, NROWS, D), jnp.float32),
            jax.ShapeDtypeStruct((47, D), jnp.float32),
        ),
        scratch_types=[
            pltpu.VMEM((2, GCHUNK, D), jnp.float32),
            pltpu.VMEM((NROWS, D), jnp.float32),
            pltpu.VMEM((TOK,), jnp.int32),
            pltpu.VMEM((TOK,), jnp.float32),
            pltpu.VMEM((8,), jnp.int32),
            pltpu.VMEM((16,), jnp.float32),
            pltpu.VMEM((D,), jnp.float32),
            pltpu.SemaphoreType.DMA,
            pltpu.SemaphoreType.DMA,
        ],
    )
    def pool(tok_hbm, w_hbm, ftok_hbm, fw_hbm, table_hbm, vals_hbm, fname_hbm,
             stage2_v, out_v, idx_v, wv_v, fidx_v, fw_v, frow_v, sem,
             osem):
        wid = lax.axis_index("s") * NC + lax.axis_index("c")

        def do_elem(i, carry):
            b = wid * per_w + i
            pltpu.sync_copy(tok_hbm.at[pl.ds(b * TOK, TOK)], idx_v)
            pltpu.sync_copy(w_hbm.at[pl.ds(b * TOK, TOK)], wv_v)
            lo0, cnt0, _ = _CHUNKS[0]
            handles = {0: pltpu.async_copy(
                table_hbm.at[idx_v.at[pl.ds(lo0, cnt0)]],
                stage2_v.at[0].at[pl.ds(0, cnt0)], sem)}
            for ci, (lo, cnt, pieces) in enumerate(_CHUNKS):
                handles[ci].wait()
                if ci + 1 < len(_CHUNKS):
                    nlo, ncnt, _ = _CHUNKS[ci + 1]
                    handles[ci + 1] = pltpu.async_copy(
                        table_hbm.at[idx_v.at[pl.ds(nlo, ncnt)]],
                        stage2_v.at[(ci + 1) % 2].at[pl.ds(0, ncnt)], sem)
                stage_v = stage2_v.at[ci % 2]
                if ci == 0:
                    @pl.when(i > 0)
                    def _():
                        pltpu.make_async_copy(
                            vals_hbm.at[b], out_v, osem).wait()
                wvec = {base: wv_v[pl.ds(base, LANES)]
                        for base in range((lo // LANES) * LANES,
                                          lo + cnt, LANES)}
                wsc = {}
                for (row, ls, n, accum) in pieces:
                    for j in range(n):
                        t = lo + ls + j
                        wsc[t] = wvec[(t // LANES) * LANES][t % LANES]

                def chunk_body(c, carry2, lo=lo, pieces=pieces, wsc=wsc,
                               stage_v=stage_v):
                    sl = pl.ds(c * LANES, LANES)
                    for (row, ls, n, accum) in pieces:
                        acc = stage_v[ls, sl] * wsc[lo + ls]
                        for j in range(1, n):
                            acc = acc + stage_v[ls + j, sl] * wsc[lo + ls + j]
                        if accum:
                            out_v[row, sl] = out_v[row, sl] + acc
                        else:
                            out_v[row, sl] = acc
                    return carry2

                lax.fori_loop(0, DCHUNKS, chunk_body, 0)
            pltpu.async_copy(out_v, vals_hbm.at[b], osem)
            return carry

        lax.fori_loop(0, per_w, do_elem, 0)
        pltpu.make_async_copy(vals_hbm.at[0], out_v, osem).wait()

        for off in (0, 32):
            row = wid + off

            @pl.when(row < 47)
            def _():
                pltpu.sync_copy(ftok_hbm.at[pl.ds(row * 8, 8)], fidx_v)
                pltpu.sync_copy(fw_hbm.at[pl.ds(row * 8, 8)],
                                fw_v.at[pl.ds(0, 8)])
                fstage = stage2_v.at[0]
                pltpu.async_copy(
                    table_hbm.at[fidx_v], fstage.at[pl.ds(0, 8)], sem).wait()
                fvec = fw_v[...]
                fsc = [fvec[j] for j in range(6)]

                def fbody(c, carry2):
                    sl = pl.ds(c * LANES, LANES)
                    acc = fstage[0, sl] * fsc[0]
                    for j in range(1, 6):
                        acc = acc + fstage[j, sl] * fsc[j]
                    frow_v[sl] = acc
                    return carry2

                lax.fori_loop(0, DCHUNKS, fbody, 0)
                pltpu.sync_copy(frow_v, fname_hbm.at[row])

    return pool(tok, wgt, ftok, fwgt, table)


def _names_fc(field_name, w_top, b_fc):
    def body(f_ref, w_ref, b_ref, o_ref):
        bf = jnp.bfloat16
        o_ref[...] = jnp.dot(
            f_ref[...].astype(bf), w_ref[...].astype(bf),
            preferred_element_type=jnp.float32) + b_ref[...][None]

    return pl.pallas_call(
        body,
        out_shape=jax.ShapeDtypeStruct((47, D), jnp.float32),
    )(field_name, w_top, b_fc)


def _fc(vals, rating, hours, names_fc, w_rating, w_hours, w_bot, w_lin):
    bsz = vals.shape[0]
    bt = 32
    bf = jnp.bfloat16

    def body(v_ref, r_ref, h_ref, nf_ref, wr_ref, wh_ref, wb_ref, wl_ref,
             o_ref):
        wb = wb_ref[...].astype(bf)
        wl = wl_ref[...].astype(bf)
        nf = nf_ref[...]

        fcv = jnp.dot(v_ref[...].astype(bf).reshape(bt * NROWS, D), wb,
                      preferred_element_type=jnp.float32)
        fcv = fcv.reshape(bt, NROWS, D) + nf[:NROWS][None]
        fcv = jnp.maximum(fcv, 0.0)
        o_ref[:, :NROWS] = jnp.dot(
            fcv.astype(bf).reshape(bt * NROWS, D), wl,
            preferred_element_type=jnp.float32).reshape(bt, NROWS, D)

        re = jnp.dot(r_ref[...].astype(bf), wr_ref[...].astype(bf),
                     preferred_element_type=jnp.float32)
        he = jnp.dot(h_ref[...].astype(bf).reshape(bt * 7, 4),
                     wh_ref[...].astype(bf),
                     preferred_element_type=jnp.float32)
        rh = jnp.concatenate([re[:, None], he.reshape(bt, 7, D)], axis=1)
        fcr = jnp.dot(rh.astype(bf).reshape(bt * 8, D), wb,
                      preferred_element_type=jnp.float32)
        fcr = fcr.reshape(bt, 8, D) + nf[NROWS:][None]
        fcr = jnp.maximum(fcr, 0.0)
        o_ref[:, NROWS:] = jnp.dot(
            fcr.astype(bf).reshape(bt * 8, D), wl,
            preferred_element_type=jnp.float32).reshape(bt, 8, D)

    return pl.pallas_call(
        body,
        grid=(bsz // bt,),
        in_specs=[
            pl.BlockSpec((bt, NROWS, D), lambda i: (i, 0, 0)),
            pl.BlockSpec((bt, 4), lambda i: (i, 0)),
            pl.BlockSpec((bt, 7, 4), lambda i: (i, 0, 0)),
            pl.BlockSpec((47, D), lambda i: (0, 0)),
            pl.BlockSpec((4, D), lambda i: (0, 0)),
            pl.BlockSpec((4, D), lambda i: (0, 0)),
            pl.BlockSpec((D, D), lambda i: (0, 0)),
            pl.BlockSpec((D, D), lambda i: (0, 0)),
        ],
        out_specs=pl.BlockSpec((bt, 47, D), lambda i: (i, 0, 0)),
        out_shape=jax.ShapeDtypeStruct((bsz, 47, D), jnp.float32),
    )(vals, rating, hours, names_fc, w_rating, w_hours, w_bot, w_lin)


def kernel(field, name, category, str_categorical, str_boolean, rating, hours,
           emb_table, W_rating, W_hours, W_fc, b_fc, W_lin):
    bsz = name.shape[0]
    f32 = jnp.float32

    nm = (name != 1).astype(f32)
    cm = (category != 1).astype(f32)
    gmask = cm.max(axis=-1)
    denom = gmask.sum(axis=-1, keepdims=True) + 1e-6
    wcat = cm * (gmask / denom)[..., None]
    scm = (str_categorical != 1).astype(f32)
    sbm = (str_boolean[..., 0] != 1).astype(f32)

    tok = jnp.concatenate(
        [name, category.reshape(bsz, 72), str_categorical.reshape(bsz, 15),
         str_boolean[..., 0]], axis=1)
    tok = jnp.pad(tok, ((0, 0), (0, TOK - 143))).astype(jnp.int32)
    wgt = jnp.concatenate(
        [nm, wcat.reshape(bsz, 72), scm.reshape(bsz, 15), sbm], axis=1)
    wgt = jnp.pad(wgt, ((0, 0), (0, TOK - 143)))

    ftok = jnp.pad(field, ((0, 0), (0, 2))).astype(jnp.int32)
    fwgt = jnp.pad((field != 1).astype(f32), ((0, 0), (0, 2)))

    vals, field_name = _sc_pool(tok.reshape(-1), wgt.reshape(-1),
                                ftok.reshape(-1), fwgt.reshape(-1), emb_table)
    names_fc = _names_fc(field_name, W_fc[:D], b_fc)
    out = _fc(vals, rating, hours, names_fc, W_rating, W_hours, W_fc[D:],
              W_lin)

    name_mask = jnp.ones((bsz, 1), dtype=bool)
    category_mask = category[:, :1, 0] != 1
    str_categorical_mask = str_categorical[:, :, 0] != 1
    str_boolean_mask = str_boolean[:, :, 0] != 1
    rating_mask = jnp.ones((bsz, 1), dtype=bool)
    hours_mask = hours.sum(axis=-1) != 0.0
    all_masks = jnp.concatenate(
        [name_mask, category_mask, str_categorical_mask, str_boolean_mask,
         rating_mask, hours_mask], axis=1)
    return out, all_masks


# speedup vs baseline: 1.6375x; 1.0851x over previous
import functools

import jax
import jax.numpy as jnp
from jax import lax
from jax.experimental import pallas as pl
from jax.experimental.pallas import tpu as pltpu
from jax.experimental.pallas import tpu_sc as plsc

D = 1024
LANES = 16
DCHUNKS = D // LANES
TOK = 160
GCHUNK = 40
NROWS = 39
NC, NS = 2, 16
NW = NC * NS


def _segments():
    segs = [(0, 0, 24), (1, 24, 72)]
    for g in range(5):
        segs.append((2 + g, 96 + 3 * g, 3))
    for t in range(32):
        segs.append((7 + t, 111 + t, 1))
    return segs


def _chunk_pieces():
    bounds = [(0, 40), (40, 80), (80, 120), (120, 144)]
    chunks = []
    for lo, hi in bounds:
        pieces = []
        for row, s, c in _segments():
            a, b = max(s, lo), min(s + c, hi)
            if a < b:
                pieces.append((row, a - lo, b - a, a > s))
        chunks.append((lo, hi - lo, pieces))
    return chunks


_CHUNKS = _chunk_pieces()


def _sc_pool(tok, wgt, ftok, fwgt, table, emit_field):
    bsz = tok.shape[0] // TOK
    per_w = bsz // NW
    mesh = plsc.VectorSubcoreMesh(core_axis_name="c", subcore_axis_name="s")
    out_type = [jax.ShapeDtypeStruct((bsz, NROWS, D), jnp.float32)]
    if emit_field:
        out_type.append(jax.ShapeDtypeStruct((47, D), jnp.float32))

    @functools.partial(
        pl.kernel,
        mesh=mesh,
        out_type=tuple(out_type),
        scratch_types=[
            pltpu.VMEM((2, GCHUNK, D), jnp.float32),
            pltpu.VMEM((NROWS, D), jnp.float32),
            pltpu.VMEM((TOK,), jnp.int32),
            pltpu.VMEM((TOK,), jnp.float32),
            pltpu.VMEM((8,), jnp.int32),
            pltpu.VMEM((16,), jnp.float32),
            pltpu.VMEM((D,), jnp.float32),
            pltpu.SemaphoreType.DMA,
            pltpu.SemaphoreType.DMA,
        ],
    )
    def pool(*refs):
        if emit_field:
            (tok_hbm, w_hbm, ftok_hbm, fw_hbm, table_hbm, vals_hbm, fname_hbm,
             stage2_v, out_v, idx_v, wv_v, fidx_v, fw_v, frow_v, sem,
             osem) = refs
        else:
            (tok_hbm, w_hbm, ftok_hbm, fw_hbm, table_hbm, vals_hbm,
             stage2_v, out_v, idx_v, wv_v, fidx_v, fw_v, frow_v, sem,
             osem) = refs
        wid = lax.axis_index("s") * NC + lax.axis_index("c")

        def do_elem(i, carry):
            b = wid * per_w + i
            pltpu.sync_copy(tok_hbm.at[pl.ds(b * TOK, TOK)], idx_v)
            pltpu.sync_copy(w_hbm.at[pl.ds(b * TOK, TOK)], wv_v)
            lo0, cnt0, _ = _CHUNKS[0]
            handles = {0: pltpu.async_copy(
                table_hbm.at[idx_v.at[pl.ds(lo0, cnt0)]],
                stage2_v.at[0].at[pl.ds(0, cnt0)], sem)}
            for ci, (lo, cnt, pieces) in enumerate(_CHUNKS):
                handles[ci].wait()
                if ci + 1 < len(_CHUNKS):
                    nlo, ncnt, _ = _CHUNKS[ci + 1]
                    handles[ci + 1] = pltpu.async_copy(
                        table_hbm.at[idx_v.at[pl.ds(nlo, ncnt)]],
                        stage2_v.at[(ci + 1) % 2].at[pl.ds(0, ncnt)], sem)
                stage_v = stage2_v.at[ci % 2]
                if ci == 0:
                    @pl.when(i > 0)
                    def _():
                        pltpu.make_async_copy(
                            vals_hbm.at[b], out_v, osem).wait()
                wvec = {base: wv_v[pl.ds(base, LANES)]
                        for base in range((lo // LANES) * LANES,
                                          lo + cnt, LANES)}
                wsc = {}
                for (row, ls, n, accum) in pieces:
                    for j in range(n):
                        t = lo + ls + j
                        wsc[t] = wvec[(t // LANES) * LANES][t % LANES]

                def chunk_body(c, carry2, lo=lo, pieces=pieces, wsc=wsc,
                               stage_v=stage_v):
                    sl = pl.ds(c * LANES, LANES)
                    for (row, ls, n, accum) in pieces:
                        acc = stage_v[ls, sl] * wsc[lo + ls]
                        for j in range(1, n):
                            acc = acc + stage_v[ls + j, sl] * wsc[lo + ls + j]
                        if accum:
                            out_v[row, sl] = out_v[row, sl] + acc
                        else:
                            out_v[row, sl] = acc
                    return carry2

                lax.fori_loop(0, DCHUNKS, chunk_body, 0)
            pltpu.async_copy(out_v, vals_hbm.at[b], osem)
            return carry

        lax.fori_loop(0, per_w, do_elem, 0)
        pltpu.make_async_copy(vals_hbm.at[0], out_v, osem).wait()

        if emit_field:
            for off in (0, 32):
                row = wid + off

                @pl.when(row < 47)
                def _():
                    pltpu.sync_copy(ftok_hbm.at[pl.ds(row * 8, 8)], fidx_v)
                    pltpu.sync_copy(fw_hbm.at[pl.ds(row * 8, 8)],
                                    fw_v.at[pl.ds(0, 8)])
                    fstage = stage2_v.at[0]
                    pltpu.async_copy(
                        table_hbm.at[fidx_v], fstage.at[pl.ds(0, 8)],
                        sem).wait()
                    fvec = fw_v[...]
                    fsc = [fvec[j] for j in range(6)]

                    def fbody(c, carry2):
                        sl = pl.ds(c * LANES, LANES)
                        acc = fstage[0, sl] * fsc[0]
                        for j in range(1, 6):
                            acc = acc + fstage[j, sl] * fsc[j]
                        frow_v[sl] = acc
                        return carry2

                    lax.fori_loop(0, DCHUNKS, fbody, 0)
                    pltpu.sync_copy(frow_v, fname_hbm.at[row])

    return pool(tok, wgt, ftok, fwgt, table)


def _names_fc(field_name, w_top, b_fc):
    def body(f_ref, w_ref, b_ref, o_ref):
        bf = jnp.bfloat16
        o_ref[...] = jnp.dot(
            f_ref[...].astype(bf), w_ref[...].astype(bf),
            preferred_element_type=jnp.float32) + b_ref[...][None]

    return pl.pallas_call(
        body,
        out_shape=jax.ShapeDtypeStruct((47, D), jnp.float32),
    )(field_name, w_top, b_fc)


def _fc(vals, rating, hours, names_fc, w_rating, w_hours, w_bot, w_lin):
    bsz = vals.shape[0]
    bt = 32
    bf = jnp.bfloat16

    def body(v_ref, r_ref, h_ref, nf_ref, wr_ref, wh_ref, wb_ref, wl_ref,
             o_ref):
        wb = wb_ref[...].astype(bf)
        wl = wl_ref[...].astype(bf)
        nf = nf_ref[...]

        fcv = jnp.dot(v_ref[...].astype(bf).reshape(bt * NROWS, D), wb,
                      preferred_element_type=jnp.float32)
        fcv = fcv.reshape(bt, NROWS, D) + nf[:NROWS][None]
        fcv = jnp.maximum(fcv, 0.0)
        o_ref[:, :NROWS] = jnp.dot(
            fcv.astype(bf).reshape(bt * NROWS, D), wl,
            preferred_element_type=jnp.float32).reshape(bt, NROWS, D)

        re = jnp.dot(r_ref[...].astype(bf), wr_ref[...].astype(bf),
                     preferred_element_type=jnp.float32)
        he = jnp.dot(h_ref[...].astype(bf).reshape(bt * 7, 4),
                     wh_ref[...].astype(bf),
                     preferred_element_type=jnp.float32)
        rh = jnp.concatenate([re[:, None], he.reshape(bt, 7, D)], axis=1)
        fcr = jnp.dot(rh.astype(bf).reshape(bt * 8, D), wb,
                      preferred_element_type=jnp.float32)
        fcr = fcr.reshape(bt, 8, D) + nf[NROWS:][None]
        fcr = jnp.maximum(fcr, 0.0)
        o_ref[:, NROWS:] = jnp.dot(
            fcr.astype(bf).reshape(bt * 8, D), wl,
            preferred_element_type=jnp.float32).reshape(bt, 8, D)

    return pl.pallas_call(
        body,
        grid=(bsz // bt,),
        in_specs=[
            pl.BlockSpec((bt, NROWS, D), lambda i: (i, 0, 0)),
            pl.BlockSpec((bt, 4), lambda i: (i, 0)),
            pl.BlockSpec((bt, 7, 4), lambda i: (i, 0, 0)),
            pl.BlockSpec((47, D), lambda i: (0, 0)),
            pl.BlockSpec((4, D), lambda i: (0, 0)),
            pl.BlockSpec((4, D), lambda i: (0, 0)),
            pl.BlockSpec((D, D), lambda i: (0, 0)),
            pl.BlockSpec((D, D), lambda i: (0, 0)),
        ],
        out_specs=pl.BlockSpec((bt, 47, D), lambda i: (i, 0, 0)),
        out_shape=jax.ShapeDtypeStruct((bsz, 47, D), jnp.float32),
    )(vals, rating, hours, names_fc, w_rating, w_hours, w_bot, w_lin)


def kernel(field, name, category, str_categorical, str_boolean, rating, hours,
           emb_table, W_rating, W_hours, W_fc, b_fc, W_lin):
    bsz = name.shape[0]
    f32 = jnp.float32

    nm = (name != 1).astype(f32)
    cm = (category != 1).astype(f32)
    gmask = cm.max(axis=-1)
    denom = gmask.sum(axis=-1, keepdims=True) + 1e-6
    wcat = cm * (gmask / denom)[..., None]
    scm = (str_categorical != 1).astype(f32)
    sbm = (str_boolean[..., 0] != 1).astype(f32)

    tok = jnp.concatenate(
        [name, category.reshape(bsz, 72), str_categorical.reshape(bsz, 15),
         str_boolean[..., 0]], axis=1)
    tok = jnp.pad(tok, ((0, 0), (0, TOK - 143))).astype(jnp.int32)
    wgt = jnp.concatenate(
        [nm, wcat.reshape(bsz, 72), scm.reshape(bsz, 15), sbm], axis=1)
    wgt = jnp.pad(wgt, ((0, 0), (0, TOK - 143)))

    ftok = jnp.pad(field, ((0, 0), (0, 2))).astype(jnp.int32)
    fwgt = jnp.pad((field != 1).astype(f32), ((0, 0), (0, 2)))

    nchunk = 4
    cs = bsz // nchunk
    ftok_f, fwgt_f = ftok.reshape(-1), fwgt.reshape(-1)
    vals_c, field_name = [], None
    for c in range(nchunk):
        sl = slice(c * cs, (c + 1) * cs)
        if c == 0:
            v, field_name = _sc_pool(tok[sl].reshape(-1), wgt[sl].reshape(-1),
                                     ftok_f, fwgt_f, emb_table, True)
        else:
            v = _sc_pool(tok[sl].reshape(-1), wgt[sl].reshape(-1),
                         ftok_f, fwgt_f, emb_table, False)
            if isinstance(v, (tuple, list)):
                v = v[0]
        vals_c.append(v)
    names_fc = _names_fc(field_name, W_fc[:D], b_fc)
    w_bot = W_fc[D:]
    out = jnp.concatenate(
        [_fc(vals_c[c], rating[c * cs:(c + 1) * cs],
             hours[c * cs:(c + 1) * cs], names_fc, W_rating, W_hours,
             w_bot, W_lin) for c in range(nchunk)], axis=0)

    name_mask = jnp.ones((bsz, 1), dtype=bool)
    category_mask = category[:, :1, 0] != 1
    str_categorical_mask = str_categorical[:, :, 0] != 1
    str_boolean_mask = str_boolean[:, :, 0] != 1
    rating_mask = jnp.ones((bsz, 1), dtype=bool)
    hours_mask = hours.sum(axis=-1) != 0.0
    all_masks = jnp.concatenate(
        [name_mask, category_mask, str_categorical_mask, str_boolean_mask,
         rating_mask, hours_mask], axis=1)
    return out, all_masks


# speedup vs baseline: 1.8552x; 1.1329x over previous
import functools

import jax
import jax.numpy as jnp
from jax import lax
from jax.experimental import pallas as pl
from jax.experimental.pallas import tpu as pltpu
from jax.experimental.pallas import tpu_sc as plsc

D = 1024
LANES = 16
DCHUNKS = D // LANES
TOK = 160
GCHUNK = 40
NROWS = 39
NC, NS = 2, 16
NW = NC * NS


def _segments():
    segs = [(0, 0, 24), (1, 24, 72)]
    for g in range(5):
        segs.append((2 + g, 96 + 3 * g, 3))
    for t in range(32):
        segs.append((7 + t, 111 + t, 1))
    return segs


def _chunk_pieces():
    bounds = [(0, 40), (40, 80), (80, 120), (120, 144)]
    chunks = []
    for lo, hi in bounds:
        pieces = []
        for row, s, c in _segments():
            a, b = max(s, lo), min(s + c, hi)
            if a < b:
                pieces.append((row, a - lo, b - a, a > s))
        chunks.append((lo, hi - lo, pieces))
    return chunks


_CHUNKS = _chunk_pieces()


def _sc_pool(tok, wgt, ftok, fwgt, table, emit_field):
    bsz = tok.shape[0] // TOK
    per_w = bsz // NW
    mesh = plsc.VectorSubcoreMesh(core_axis_name="c", subcore_axis_name="s")
    out_type = [jax.ShapeDtypeStruct((bsz, NROWS, D), jnp.float32)]
    if emit_field:
        out_type.append(jax.ShapeDtypeStruct((47, D), jnp.float32))

    @functools.partial(
        pl.kernel,
        mesh=mesh,
        out_type=tuple(out_type),
        scratch_types=[
            pltpu.VMEM((2, GCHUNK, D), jnp.float32),
            pltpu.VMEM((NROWS, D), jnp.float32),
            pltpu.VMEM((TOK,), jnp.int32),
            pltpu.VMEM((TOK,), jnp.float32),
            pltpu.VMEM((8,), jnp.int32),
            pltpu.VMEM((16,), jnp.float32),
            pltpu.VMEM((D,), jnp.float32),
            pltpu.SemaphoreType.DMA,
            pltpu.SemaphoreType.DMA,
        ],
    )
    def pool(*refs):
        if emit_field:
            (tok_hbm, w_hbm, ftok_hbm, fw_hbm, table_hbm, vals_hbm, fname_hbm,
             stage2_v, out_v, idx_v, wv_v, fidx_v, fw_v, frow_v, sem,
             osem) = refs
        else:
            (tok_hbm, w_hbm, ftok_hbm, fw_hbm, table_hbm, vals_hbm,
             stage2_v, out_v, idx_v, wv_v, fidx_v, fw_v, frow_v, sem,
             osem) = refs
        wid = lax.axis_index("s") * NC + lax.axis_index("c")

        def do_elem(i, carry):
            b = wid * per_w + i
            pltpu.sync_copy(tok_hbm.at[pl.ds(b * TOK, TOK)], idx_v)
            pltpu.sync_copy(w_hbm.at[pl.ds(b * TOK, TOK)], wv_v)
            lo0, cnt0, _ = _CHUNKS[0]
            handles = {0: pltpu.async_copy(
                table_hbm.at[idx_v.at[pl.ds(lo0, cnt0)]],
                stage2_v.at[0].at[pl.ds(0, cnt0)], sem)}
            for ci, (lo, cnt, pieces) in enumerate(_CHUNKS):
                handles[ci].wait()
                if ci + 1 < len(_CHUNKS):
                    nlo, ncnt, _ = _CHUNKS[ci + 1]
                    handles[ci + 1] = pltpu.async_copy(
                        table_hbm.at[idx_v.at[pl.ds(nlo, ncnt)]],
                        stage2_v.at[(ci + 1) % 2].at[pl.ds(0, ncnt)], sem)
                stage_v = stage2_v.at[ci % 2]
                if ci == 0:
                    @pl.when(i > 0)
                    def _():
                        pltpu.make_async_copy(
                            vals_hbm.at[b], out_v, osem).wait()
                wvec = {base: wv_v[pl.ds(base, LANES)]
                        for base in range((lo // LANES) * LANES,
                                          lo + cnt, LANES)}
                wsc = {}
                for (row, ls, n, accum) in pieces:
                    for j in range(n):
                        t = lo + ls + j
                        wsc[t] = wvec[(t // LANES) * LANES][t % LANES]

                def chunk_body(c, carry2, lo=lo, pieces=pieces, wsc=wsc,
                               stage_v=stage_v):
                    sl = pl.ds(c * LANES, LANES)
                    for (row, ls, n, accum) in pieces:
                        acc = stage_v[ls, sl] * wsc[lo + ls]
                        for j in range(1, n):
                            acc = acc + stage_v[ls + j, sl] * wsc[lo + ls + j]
                        if accum:
                            out_v[row, sl] = out_v[row, sl] + acc
                        else:
                            out_v[row, sl] = acc
                    return carry2

                lax.fori_loop(0, DCHUNKS, chunk_body, 0)
            pltpu.async_copy(out_v, vals_hbm.at[b], osem)
            return carry

        lax.fori_loop(0, per_w, do_elem, 0)
        pltpu.make_async_copy(vals_hbm.at[0], out_v, osem).wait()

        if emit_field:
            for off in (0, 32):
                row = wid + off

                @pl.when(row < 47)
                def _():
                    pltpu.sync_copy(ftok_hbm.at[pl.ds(row * 8, 8)], fidx_v)
                    pltpu.sync_copy(fw_hbm.at[pl.ds(row * 8, 8)],
                                    fw_v.at[pl.ds(0, 8)])
                    fstage = stage2_v.at[0]
                    pltpu.async_copy(
                        table_hbm.at[fidx_v], fstage.at[pl.ds(0, 8)],
                        sem).wait()
                    fvec = fw_v[...]
                    fsc = [fvec[j] for j in range(6)]

                    def fbody(c, carry2):
                        sl = pl.ds(c * LANES, LANES)
                        acc = fstage[0, sl] * fsc[0]
                        for j in range(1, 6):
                            acc = acc + fstage[j, sl] * fsc[j]
                        frow_v[sl] = acc
                        return carry2

                    lax.fori_loop(0, DCHUNKS, fbody, 0)
                    pltpu.sync_copy(frow_v, fname_hbm.at[row])

    return pool(tok, wgt, ftok, fwgt, table)


def _names_fc(field_name, w_top, b_fc):
    def body(f_ref, w_ref, b_ref, o_ref):
        bf = jnp.bfloat16
        o_ref[...] = jnp.dot(
            f_ref[...].astype(bf), w_ref[...].astype(bf),
            preferred_element_type=jnp.float32) + b_ref[...][None]

    return pl.pallas_call(
        body,
        out_shape=jax.ShapeDtypeStruct((47, D), jnp.float32),
    )(field_name, w_top, b_fc)


def _fc(vals, rating, hours, names_fc, w_rating, w_hours, w_bot, w_lin,
        acc, chunk, nchunk):
    csz = vals.shape[0]
    bsz = csz * nchunk
    bt = 32
    bf = jnp.bfloat16

    def body(v_ref, r_ref, h_ref, nf_ref, wr_ref, wh_ref, wb_ref, wl_ref,
             *rest):
        o_ref = rest[-1]
        wb = wb_ref[...].astype(bf)
        wl = wl_ref[...].astype(bf)
        nf = nf_ref[...]

        fcv = jnp.dot(v_ref[...].astype(bf).reshape(bt * NROWS, D), wb,
                      preferred_element_type=jnp.float32)
        fcv = fcv.reshape(bt, NROWS, D) + nf[:NROWS][None]
        fcv = jnp.maximum(fcv, 0.0)
        o_ref[:, :NROWS] = jnp.dot(
            fcv.astype(bf).reshape(bt * NROWS, D), wl,
            preferred_element_type=jnp.float32).reshape(bt, NROWS, D)

        re = jnp.dot(r_ref[...].astype(bf), wr_ref[...].astype(bf),
                     preferred_element_type=jnp.float32)
        he = jnp.dot(h_ref[...].astype(bf).reshape(bt * 7, 4),
                     wh_ref[...].astype(bf),
                     preferred_element_type=jnp.float32)
        rh = jnp.concatenate([re[:, None], he.reshape(bt, 7, D)], axis=1)
        fcr = jnp.dot(rh.astype(bf).reshape(bt * 8, D), wb,
                      preferred_element_type=jnp.float32)
        fcr = fcr.reshape(bt, 8, D) + nf[NROWS:][None]
        fcr = jnp.maximum(fcr, 0.0)
        o_ref[:, NROWS:] = jnp.dot(
            fcr.astype(bf).reshape(bt * 8, D), wl,
            preferred_element_type=jnp.float32).reshape(bt, 8, D)

    off = chunk * (csz // bt)
    in_specs = [
        pl.BlockSpec((bt, NROWS, D), lambda i: (i, 0, 0)),
        pl.BlockSpec((bt, 4), lambda i: (i, 0)),
        pl.BlockSpec((bt, 7, 4), lambda i: (i, 0, 0)),
        pl.BlockSpec((47, D), lambda i: (0, 0)),
        pl.BlockSpec((4, D), lambda i: (0, 0)),
        pl.BlockSpec((4, D), lambda i: (0, 0)),
        pl.BlockSpec((D, D), lambda i: (0, 0)),
        pl.BlockSpec((D, D), lambda i: (0, 0)),
    ]
    args = [vals, rating, hours, names_fc, w_rating, w_hours, w_bot, w_lin]
    aliases = {}
    if acc is not None:
        in_specs.append(pl.BlockSpec(memory_space=pl.ANY))
        args.append(acc)
        aliases = {8: 0}
    return pl.pallas_call(
        body,
        grid=(csz // bt,),
        in_specs=in_specs,
        out_specs=pl.BlockSpec((bt, 47, D), lambda i: (off + i, 0, 0)),
        out_shape=jax.ShapeDtypeStruct((bsz, 47, D), jnp.float32),
        input_output_aliases=aliases,
    )(*args)


def kernel(field, name, category, str_categorical, str_boolean, rating, hours,
           emb_table, W_rating, W_hours, W_fc, b_fc, W_lin):
    bsz = name.shape[0]
    f32 = jnp.float32

    nm = (name != 1).astype(f32)
    cm = (category != 1).astype(f32)
    gmask = cm.max(axis=-1)
    denom = gmask.sum(axis=-1, keepdims=True) + 1e-6
    wcat = cm * (gmask / denom)[..., None]
    scm = (str_categorical != 1).astype(f32)
    sbm = (str_boolean[..., 0] != 1).astype(f32)

    tok = jnp.concatenate(
        [name, category.reshape(bsz, 72), str_categorical.reshape(bsz, 15),
         str_boolean[..., 0]], axis=1)
    tok = jnp.pad(tok, ((0, 0), (0, TOK - 143))).astype(jnp.int32)
    wgt = jnp.concatenate(
        [nm, wcat.reshape(bsz, 72), scm.reshape(bsz, 15), sbm], axis=1)
    wgt = jnp.pad(wgt, ((0, 0), (0, TOK - 143)))

    ftok = jnp.pad(field, ((0, 0), (0, 2))).astype(jnp.int32)
    fwgt = jnp.pad((field != 1).astype(f32), ((0, 0), (0, 2)))

    nchunk = 4
    cs = bsz // nchunk
    ftok_f, fwgt_f = ftok.reshape(-1), fwgt.reshape(-1)
    vals_c, field_name = [], None
    for c in range(nchunk):
        sl = slice(c * cs, (c + 1) * cs)
        if c == 0:
            v, field_name = _sc_pool(tok[sl].reshape(-1), wgt[sl].reshape(-1),
                                     ftok_f, fwgt_f, emb_table, True)
        else:
            v = _sc_pool(tok[sl].reshape(-1), wgt[sl].reshape(-1),
                         ftok_f, fwgt_f, emb_table, False)
            if isinstance(v, (tuple, list)):
                v = v[0]
        vals_c.append(v)
    names_fc = _names_fc(field_name, W_fc[:D], b_fc)
    w_bot = W_fc[D:]
    out = None
    for c in range(nchunk):
        out = _fc(vals_c[c], rating[c * cs:(c + 1) * cs],
                  hours[c * cs:(c + 1) * cs], names_fc, W_rating, W_hours,
                  w_bot, W_lin, out, c, nchunk)

    name_mask = jnp.ones((bsz, 1), dtype=bool)
    category_mask = category[:, :1, 0] != 1
    str_categorical_mask = str_categorical[:, :, 0] != 1
    str_boolean_mask = str_boolean[:, :, 0] != 1
    rating_mask = jnp.ones((bsz, 1), dtype=bool)
    hours_mask = hours.sum(axis=-1) != 0.0
    all_masks = jnp.concatenate(
        [name_mask, category_mask, str_categorical_mask, str_boolean_mask,
         rating_mask, hours_mask], axis=1)
    return out, all_masks
